# baseline (speedup 1.0000x reference)
import functools

import jax
import jax.numpy as jnp
from jax import lax
from jax.experimental import pallas as pl
from jax.experimental.pallas import tpu as pltpu
from jax.experimental.pallas import tpu_sc as plsc

VOCAB = 1000000
D = 32
B, T = 4096, 200
TOTAL = B * T
NW = 32
PER_W = TOTAL // NW
CHUNK = 1024
G = 128
SUBG = CHUNK // G
N_CHUNKS = PER_W // CHUNK

_LANES = 16
_UNROLL = 8


def _tanh16(x):
    t = jnp.exp(x * -2.0)
    return 2.0 / (1.0 + t) - 1.0


def _body(table_hbm, idx_hbm, out_hbm, idx_v, rows_v, gsem, ssem):
    wid = lax.axis_index("s") * 2 + lax.axis_index("c")
    w_base = wid * PER_W
    w_irow = wid * (PER_W // G)

    def load_chunk(c, buf):
        pltpu.sync_copy(idx_hbm.at[pl.ds(w_irow + c * SUBG, SUBG)],
                        idx_v.at[buf])
        for j in range(SUBG):
            pltpu.async_copy(
                table_hbm.at[idx_v.at[buf, j]],
                rows_v.at[buf, pl.ds(j * G, G)],
                gsem,
            )

    def wait_gathers(c, buf):
        pltpu.make_async_copy(
            out_hbm.at[pl.ds(w_base + c * CHUNK, CHUNK)],
            rows_v.at[buf], gsem,
        ).wait()

    def store_chunk(c, buf):
        pltpu.async_copy(
            rows_v.at[buf],
            out_hbm.at[pl.ds(w_base + c * CHUNK, CHUNK)], ssem,
        )

    def wait_store(c, buf):
        pltpu.make_async_copy(
            rows_v.at[buf],
            out_hbm.at[pl.ds(w_base + c * CHUNK, CHUNK)], ssem,
        ).wait()

    def compute(buf):
        def row_step(i, _):
            r0 = i * _UNROLL
            for u in range(_UNROLL):
                for h in range(D // _LANES):
                    sl = pl.ds(h * _LANES, _LANES)
                    rows_v[buf, r0 + u, sl] = _tanh16(rows_v[buf, r0 + u, sl])
            return 0

        lax.fori_loop(0, CHUNK // _UNROLL, row_step, 0)

    load_chunk(0, 0)

    def step(c, _):
        cur = lax.rem(c, 2)
        nxt = 1 - cur

        @pl.when(c + 1 < N_CHUNKS)
        def _():
            @pl.when(c >= 1)
            def _():
                wait_store(c - 1, nxt)

            load_chunk(c + 1, nxt)

        wait_gathers(c, cur)
        compute(cur)
        store_chunk(c, cur)
        return 0

    lax.fori_loop(0, N_CHUNKS, step, 0)
    wait_store(N_CHUNKS - 2, (N_CHUNKS - 2) % 2)
    wait_store(N_CHUNKS - 1, (N_CHUNKS - 1) % 2)


@jax.jit
def kernel(input_ids, table):
    idx = input_ids.astype(jnp.int32).reshape(TOTAL // G, G)
    mesh = plsc.VectorSubcoreMesh(core_axis_name="c", subcore_axis_name="s")
    out = pl.kernel(
        _body,
        out_type=jax.ShapeDtypeStruct((TOTAL, D), jnp.float32),
        mesh=mesh,
        compiler_params=pltpu.CompilerParams(use_tc_tiling_on_sc=False),
        scratch_types=[
            pltpu.VMEM((2, SUBG, G), jnp.int32),
            pltpu.VMEM((2, CHUNK, D), jnp.float32),
            pltpu.SemaphoreType.DMA,
            pltpu.SemaphoreType.DMA,
        ],
    )(table, idx)
    return out.reshape(B, T, D)


# speedup vs baseline: 1.9638x; 1.9638x over previous
import functools

import jax
import jax.numpy as jnp
from jax import lax
from jax.experimental import pallas as pl
from jax.experimental.pallas import tpu as pltpu
from jax.experimental.pallas import tpu_sc as plsc

VOCAB = 1000000
D = 32
B, T = 4096, 200
TOTAL = B * T
NW = 32
PER_W = TOTAL // NW
CHUNK = 1024
G = 128
SUBG = CHUNK // G
N_CHUNKS = PER_W // CHUNK

_LANES = 16
_UNROLL = 8


def _tanh16(x):
    t = jnp.exp(x * -2.0)
    return 2.0 / (1.0 + t) - 1.0


def _body(table_hbm, idx_hbm, out_hbm, idx_v, rows_v,
          g0, g1, g2, s0, s1, s2):
    gs = (g0, g1, g2)
    ss = (s0, s1, s2)
    wid = lax.axis_index("s") * 2 + lax.axis_index("c")
    w_base = wid * PER_W
    w_irow = wid * (PER_W // G)

    def load_chunk(c, b):
        pltpu.sync_copy(idx_hbm.at[pl.ds(w_irow + c * SUBG, SUBG)],
                        idx_v.at[b])
        for j in range(SUBG):
            pltpu.async_copy(
                table_hbm.at[idx_v.at[b, j]],
                rows_v.at[b, pl.ds(j * G, G)],
                gs[b],
            )

    def wait_gathers(c, b):
        pltpu.make_async_copy(
            out_hbm.at[pl.ds(w_base + c * CHUNK, CHUNK)],
            rows_v.at[b], gs[b],
        ).wait()

    def store_chunk(c, b):
        pltpu.async_copy(
            rows_v.at[b],
            out_hbm.at[pl.ds(w_base + c * CHUNK, CHUNK)], ss[b],
        )

    def wait_store(c, b):
        pltpu.make_async_copy(
            rows_v.at[b],
            out_hbm.at[pl.ds(w_base + c * CHUNK, CHUNK)], ss[b],
        ).wait()

    def compute(b):
        def row_step(i, _):
            r0 = i * _UNROLL
            for u in range(_UNROLL):
                for h in range(D // _LANES):
                    sl = pl.ds(h * _LANES, _LANES)
                    rows_v[b, r0 + u, sl] = _tanh16(rows_v[b, r0 + u, sl])
            return 0

        lax.fori_loop(0, CHUNK // _UNROLL, row_step, 0)

    def substep(c, b, bn):
        @pl.when(c >= 2)
        def _():
            wait_store(c - 2, bn)

        load_chunk(c + 1, bn)
        wait_gathers(c, b)
        compute(b)
        store_chunk(c, b)

    load_chunk(0, 0)

    def trip(k, _):
        c0 = k * 3
        substep(c0, 0, 1)
        substep(c0 + 1, 1, 2)
        substep(c0 + 2, 2, 0)
        return 0

    lax.fori_loop(0, (N_CHUNKS - 1) // 3, trip, 0)
    wait_store(N_CHUNKS - 3, 1)
    wait_gathers(N_CHUNKS - 1, 0)
    compute(0)
    store_chunk(N_CHUNKS - 1, 0)
    wait_store(N_CHUNKS - 2, 2)
    wait_store(N_CHUNKS - 1, 0)


@jax.jit
def kernel(input_ids, table):
    idx = input_ids.astype(jnp.int32).reshape(TOTAL // G, G)
    mesh = plsc.VectorSubcoreMesh(core_axis_name="c", subcore_axis_name="s")
    out = pl.kernel(
        _body,
        out_type=jax.ShapeDtypeStruct((TOTAL, D), jnp.float32),
        mesh=mesh,
        compiler_params=pltpu.CompilerParams(use_tc_tiling_on_sc=False),
        scratch_types=[
            pltpu.VMEM((3, SUBG, G), jnp.int32),
            pltpu.VMEM((3, CHUNK, D), jnp.float32),
        ] + [pltpu.SemaphoreType.DMA] * 6,
    )(table, idx)
    return out.reshape(B, T, D)
